# Initial kernel scaffold; baseline (speedup 1.0000x reference)
import jax
import jax.numpy as jnp
from jax.experimental import pallas as pl


def kernel(x, edge_index, batch, W_proj, b_proj, Wc, bc, W_cls, b_cls):
    raise NotImplementedError("write your pallas kernel here")


# speedup vs baseline: 6.3121x; 6.3121x over previous
import functools

import jax
import jax.numpy as jnp
from jax import lax
from jax.experimental import pallas as pl
from jax.experimental.pallas import tpu as pltpu
from jax.experimental.pallas import tpu_sc as plsc

N = 10000
E = 320000
G = 64
D = 128
NUM_LAYERS = 4

NC = 2
NS = 16
NW = NC * NS
FPT = D // NW
EPT = E // NW
CH = 2000
NCH = E // CH
PCH = EPT // CH
NB = 2048
NBLK = -(-N // NB)

_mesh = plsc.VectorSubcoreMesh(core_axis_name="c", subcore_axis_name="s")
_sc_params = pltpu.CompilerParams(needs_layout_passes=False)


@functools.partial(
    pl.kernel,
    out_type=(jax.ShapeDtypeStruct((E,), jnp.int32),
              jax.ShapeDtypeStruct((NW * N,), jnp.float32)),
    mesh=_mesh,
    compiler_params=_sc_params,
    scratch_types=[pltpu.VMEM((N,), jnp.float32),
                   pltpu.VMEM((CH,), jnp.int32),
                   pltpu.VMEM((CH,), jnp.int32),
                   pltpu.VMEM((CH,), jnp.int32)],
)
def _edge_prep(edge_hbm, packed_hbm, degp_hbm, deg, srcb, dstb, pkb):
    wid = lax.axis_index("s") * NC + lax.axis_index("c")
    zero16 = jnp.zeros((16,), jnp.float32)

    def zbody(i, c):
        deg[pl.ds(i * 16, 16)] = zero16
        return c

    lax.fori_loop(0, N // 16, zbody, 0)

    ones16 = jnp.full((16,), 1.0, jnp.float32)
    base = wid * EPT
    for ci in range(PCH):
        off = base + ci * CH
        pltpu.sync_copy(edge_hbm.at[pl.ds(off, CH)], srcb)
        pltpu.sync_copy(edge_hbm.at[pl.ds(E + off, CH)], dstb)

        def gbody(g, c):
            s = srcb[pl.ds(g * 16, 16)]
            d = dstb[pl.ds(g * 16, 16)]
            pkb[pl.ds(g * 16, 16)] = (s << 14) | d
            plsc.addupdate_scatter(deg, (d,), ones16)
            return c

        lax.fori_loop(0, CH // 16, gbody, 0)
        pltpu.sync_copy(pkb, packed_hbm.at[pl.ds(off, CH)])
    pltpu.sync_copy(deg, degp_hbm.at[pl.ds(wid * N, N)])


@functools.partial(
    pl.kernel,
    out_type=jax.ShapeDtypeStruct((D * N,), jnp.float32),
    mesh=_mesh,
    compiler_params=_sc_params,
    scratch_types=[pltpu.VMEM((FPT * N,), jnp.float32),
                   pltpu.VMEM((FPT * N,), jnp.float32),
                   pltpu.VMEM((CH,), jnp.int32),
                   pltpu.VMEM((CH,), jnp.int32),
                   pltpu.SemaphoreType.DMA,
                   pltpu.SemaphoreType.DMA],
)
def _aggregate(ut_hbm, packed_hbm, acc_hbm, ut, acc, eb0, eb1, sem0, sem1):
    wid = lax.axis_index("s") * NC + lax.axis_index("c")
    fbase = wid * (FPT * N)
    pltpu.sync_copy(ut_hbm.at[pl.ds(fbase, FPT * N)], ut)

    zero16 = jnp.zeros((16,), jnp.float32)

    def zbody(i, c):
        acc[pl.ds(i * 16, 16)] = zero16
        return c

    lax.fori_loop(0, FPT * N // 16, zbody, 0)

    ebs = (eb0, eb1)
    sems = (sem0, sem1)
    pltpu.async_copy(packed_hbm.at[pl.ds(0, CH)], eb0, sem0)
    pltpu.async_copy(packed_hbm.at[pl.ds(CH, CH)], eb1, sem1)

    def process(eb):
        def gbody(g, c):
            w = eb[pl.ds(g * 16, 16)]
            s = w >> 14
            d = w & 16383
            for f in range(FPT):
                o = f * N
                uv = plsc.load_gather(ut, (s + o,))
                plsc.addupdate_scatter(acc, (d + o,), uv)
            return c

        lax.fori_loop(0, CH // 16, gbody, 0)

    def cbody(c2, carry):
        for b in range(2):
            ci = c2 * 2 + b
            pltpu.make_async_copy(
                packed_hbm.at[pl.ds(ci * CH, CH)], ebs[b], sems[b]).wait()
            process(ebs[b])

            @pl.when(ci + 2 < NCH)
            def _():
                pltpu.async_copy(
                    packed_hbm.at[pl.ds((ci + 2) * CH, CH)], ebs[b], sems[b])
        return carry

    lax.fori_loop(0, NCH // 2, cbody, 0)
    pltpu.sync_copy(acc, acc_hbm.at[pl.ds(fbase, FPT * N)])


def _dot(a, b, dims):
    return lax.dot_general(a, b, (dims, ((), ())),
                           preferred_element_type=jnp.float32)


def _prep_body(x_ref, wp_ref, bp_ref, wc0_ref, degp_ref, ut_ref, dis_ref):
    deg = jnp.sum(degp_ref[...], axis=0, keepdims=True) + 1.0
    dis = lax.rsqrt(deg)
    dis_ref[...] = dis
    h = jax.nn.relu(_dot(wp_ref[...], x_ref[...], ((0,), (1,))) + bp_ref[...])
    ut_ref[...] = dis * _dot(wc0_ref[...], h, ((0,), (0,)))


def _layer_body(acc_ref, ut_ref, dis_ref, bc_ref, wc_ref, out_ref):
    dis = dis_ref[...]
    h = jax.nn.relu(dis * (acc_ref[...] + ut_ref[...]) + bc_ref[...])
    out_ref[...] = dis * _dot(wc_ref[...], h, ((0,), (0,)))


def _final_body(acc_ref, ut_ref, dis_ref, bc_ref, batch_ref, wcls_ref,
                bcls_ref, out_ref, sums_ref, cnt_ref):
    i = pl.program_id(0)

    @pl.when(i == 0)
    def _():
        sums_ref[...] = jnp.zeros_like(sums_ref)
        cnt_ref[...] = jnp.zeros_like(cnt_ref)

    dis = dis_ref[...]
    h = jax.nn.relu(dis * (acc_ref[...] + ut_ref[...]) + bc_ref[...])
    gid = lax.broadcasted_iota(jnp.int32, (G, NB), 0)
    col = lax.broadcasted_iota(jnp.int32, (G, NB), 1) + i * NB
    sel = jnp.where((gid == batch_ref[...]) & (col < N), 1.0, 0.0)
    sums_ref[...] += _dot(sel, h, ((1,), (1,)))
    cnt_ref[...] += jnp.sum(sel, axis=1, keepdims=True)

    @pl.when(i == NBLK - 1)
    def _():
        pooled = sums_ref[...] / jnp.maximum(cnt_ref[...], 1.0)
        out_ref[...] = _dot(pooled, wcls_ref[...], ((1,), (0,))) + bcls_ref[...]


def _tc_prep(x, w_proj, b_proj, wc0, degp):
    return pl.pallas_call(
        _prep_body,
        grid=(NBLK,),
        in_specs=[
            pl.BlockSpec((NB, D), lambda i: (i, 0)),
            pl.BlockSpec((D, D), lambda i: (0, 0)),
            pl.BlockSpec((D, 1), lambda i: (0, 0)),
            pl.BlockSpec((D, D), lambda i: (0, 0)),
            pl.BlockSpec((NW, NB), lambda i: (0, i)),
        ],
        out_specs=[
            pl.BlockSpec((D, NB), lambda i: (0, i)),
            pl.BlockSpec((1, NB), lambda i: (0, i)),
        ],
        out_shape=[
            jax.ShapeDtypeStruct((D, N), jnp.float32),
            jax.ShapeDtypeStruct((1, N), jnp.float32),
        ],
    )(x, w_proj, b_proj, wc0, degp)


def _tc_layer(acc, ut, dis, bc, wc):
    return pl.pallas_call(
        _layer_body,
        grid=(NBLK,),
        in_specs=[
            pl.BlockSpec((D, NB), lambda i: (0, i)),
            pl.BlockSpec((D, NB), lambda i: (0, i)),
            pl.BlockSpec((1, NB), lambda i: (0, i)),
            pl.BlockSpec((D, 1), lambda i: (0, 0)),
            pl.BlockSpec((D, D), lambda i: (0, 0)),
        ],
        out_specs=pl.BlockSpec((D, NB), lambda i: (0, i)),
        out_shape=jax.ShapeDtypeStruct((D, N), jnp.float32),
    )(acc, ut, dis, bc, wc)


def _tc_final(acc, ut, dis, bc, batch, w_cls, b_cls):
    return pl.pallas_call(
        _final_body,
        grid=(NBLK,),
        in_specs=[
            pl.BlockSpec((D, NB), lambda i: (0, i)),
            pl.BlockSpec((D, NB), lambda i: (0, i)),
            pl.BlockSpec((1, NB), lambda i: (0, i)),
            pl.BlockSpec((D, 1), lambda i: (0, 0)),
            pl.BlockSpec((1, NB), lambda i: (0, i)),
            pl.BlockSpec((D, 10), lambda i: (0, 0)),
            pl.BlockSpec((1, 10), lambda i: (0, 0)),
        ],
        out_specs=pl.BlockSpec((G, 10), lambda i: (0, 0)),
        out_shape=jax.ShapeDtypeStruct((G, 10), jnp.float32),
        scratch_shapes=[pltpu.VMEM((G, D), jnp.float32),
                        pltpu.VMEM((G, 1), jnp.float32)],
    )(acc, ut, dis, bc, batch, w_cls, b_cls)


def kernel(x, edge_index, batch, W_proj, b_proj, Wc, bc, W_cls, b_cls):
    packed, degp = _edge_prep(edge_index.reshape(2 * E))
    degp = degp.reshape(NW, N)

    ut, dis = _tc_prep(x, W_proj, b_proj.reshape(D, 1), Wc[0], degp)
    for i in range(NUM_LAYERS):
        acc = _aggregate(ut.reshape(D * N), packed).reshape(D, N)
        if i < NUM_LAYERS - 1:
            ut = _tc_layer(acc, ut, dis, bc[i].reshape(D, 1), Wc[i + 1])
        else:
            out = _tc_final(acc, ut, dis, bc[i].reshape(D, 1),
                            batch.reshape(1, N), W_cls, b_cls.reshape(1, 10))
    return out
